# baseline (speedup 1.0000x reference)
import functools

import jax
import jax.numpy as jnp
from jax import lax
from jax.experimental import pallas as pl
from jax.experimental.pallas import tpu as pltpu
from jax.experimental.pallas import tpu_sc as plsc

_PS_DIM = 64
_TABLE_ROWS = 200
_LANES = 16


@functools.lru_cache(maxsize=None)
def _gather_call(total_rows, chunk):
    info = plsc.get_sparse_core_info()
    nw = info.num_cores * info.num_subcores
    per_w = total_rows // nw
    n_chunks = per_w // chunk
    assert per_w * nw == total_rows and n_chunks * chunk == per_w

    mesh = plsc.VectorSubcoreMesh(core_axis_name="c", subcore_axis_name="s")

    @functools.partial(
        pl.kernel,
        mesh=mesh,
        compiler_params=pltpu.CompilerParams(needs_layout_passes=False),
        out_type=jax.ShapeDtypeStruct((total_rows * _PS_DIM,), jnp.float32),
        scratch_types=[
            pltpu.VMEM((_TABLE_ROWS * _PS_DIM,), jnp.float32),
            pltpu.VMEM((chunk,), jnp.int32),
            pltpu.VMEM((chunk * _PS_DIM,), jnp.float32),
        ],
    )
    def k(table_hbm, idx_hbm, out_hbm, table_v, idx_v, rows_v):
        wid = lax.axis_index("s") * info.num_cores + lax.axis_index("c")
        base = wid * per_w
        pltpu.sync_copy(table_hbm, table_v)
        lane = lax.iota(jnp.int32, _LANES)

        def chunk_body(g, carry):
            row0 = base + g * chunk
            pltpu.sync_copy(idx_hbm.at[pl.ds(row0, chunk)], idx_v)

            def j_body(j, c):
                fb = idx_v[pl.ds(j * _LANES, _LANES)] * _PS_DIM
                ob = lane * _PS_DIM + j * (_LANES * _PS_DIM)
                col = lane
                for k in range(_PS_DIM):
                    v = plsc.load_gather(table_v, [fb + col])
                    plsc.store_scatter(rows_v, [ob + col], v)
                    if k != _PS_DIM - 1:
                        col = (col + 1) & (_PS_DIM - 1)
                return c

            lax.fori_loop(0, chunk // _LANES, j_body, 0, unroll=False)
            pltpu.sync_copy(
                rows_v, out_hbm.at[pl.ds(row0 * _PS_DIM, chunk * _PS_DIM)]
            )
            return carry

        lax.fori_loop(0, n_chunks, chunk_body, 0, unroll=False)

    return k


def kernel(batch_rgn_sqn, encoding):
    b, l = batch_rgn_sqn.shape
    table = encoding[:_TABLE_ROWS, :_PS_DIM].reshape(-1)
    idx = batch_rgn_sqn.reshape(-1).astype(jnp.int32)
    out = _gather_call(b * l, 1024)(table, idx)
    return out.reshape(b, l, _PS_DIM)


# speedup vs baseline: 1.1502x; 1.1502x over previous
import functools

import jax
import jax.numpy as jnp
from jax import lax
from jax.experimental import pallas as pl
from jax.experimental.pallas import tpu as pltpu
from jax.experimental.pallas import tpu_sc as plsc

_PS_DIM = 64
_TABLE_ROWS = 200
_LANES = 16


@functools.lru_cache(maxsize=None)
def _gather_call(total_rows, chunk):
    info = plsc.get_sparse_core_info()
    nw = info.num_cores * info.num_subcores
    per_w = total_rows // nw
    n_chunks = per_w // chunk
    assert per_w * nw == total_rows and n_chunks * chunk == per_w
    assert n_chunks % 2 == 0

    mesh = plsc.VectorSubcoreMesh(core_axis_name="c", subcore_axis_name="s")

    @functools.partial(
        pl.kernel,
        mesh=mesh,
        compiler_params=pltpu.CompilerParams(needs_layout_passes=False),
        out_type=jax.ShapeDtypeStruct((total_rows * _PS_DIM,), jnp.float32),
        scratch_types=[
            pltpu.VMEM((_TABLE_ROWS * _PS_DIM,), jnp.float32),
            pltpu.VMEM((chunk,), jnp.int32),
            pltpu.VMEM((chunk,), jnp.int32),
            pltpu.VMEM((chunk * _PS_DIM,), jnp.float32),
            pltpu.VMEM((chunk * _PS_DIM,), jnp.float32),
            pltpu.SemaphoreType.DMA,
            pltpu.SemaphoreType.DMA,
            pltpu.SemaphoreType.DMA,
            pltpu.SemaphoreType.DMA,
        ],
    )
    def k(table_hbm, idx_hbm, out_hbm, table_v, idx_v0, idx_v1, rows_v0,
          rows_v1, isem0, isem1, osem0, osem1):
        wid = lax.axis_index("s") * info.num_cores + lax.axis_index("c")
        base = wid * per_w
        pltpu.sync_copy(table_hbm, table_v)
        idx_bufs = (idx_v0, idx_v1)
        rows_bufs = (rows_v0, rows_v1)
        isems = (isem0, isem1)
        osems = (osem0, osem1)
        nq = _PS_DIM // _LANES

        def idx_start(g, par):
            return pltpu.make_async_copy(
                idx_hbm.at[pl.ds(base + g * chunk, chunk)], idx_bufs[par],
                isems[par],
            )

        def out_copy(g, par):
            return pltpu.make_async_copy(
                rows_bufs[par],
                out_hbm.at[pl.ds((base + g * chunk) * _PS_DIM,
                                 chunk * _PS_DIM)],
                osems[par],
            )

        def compose(par):
            idx_v = idx_bufs[par]
            rows_v = rows_bufs[par]

            def j_body(j, c):
                r0 = j * _LANES
                idxv = idx_v[pl.ds(r0, _LANES)] * _PS_DIM
                for u in range(_LANES):
                    off = idxv[u]
                    ob = (r0 + u) * _PS_DIM
                    for q in range(nq):
                        rows_v[pl.ds(ob + q * _LANES, _LANES)] = (
                            table_v[pl.ds(off + q * _LANES, _LANES)]
                        )
                return c

            lax.fori_loop(0, chunk // _LANES, j_body, 0, unroll=False)

        idx_start(0, 0).start()
        idx_start(1, 1).start()
        idx_start(0, 0).wait()
        compose(0)
        out_copy(0, 0).start()

        def pair_body(g2, c):
            g = g2 * 2 + 1
            idx_start(g, 1).wait()
            idx_start(g + 1, 0).start()

            @pl.when(g2 > 0)
            def _():
                out_copy(g - 2, 1).wait()

            compose(1)
            out_copy(g, 1).start()
            idx_start(g + 1, 0).wait()

            @pl.when(g + 2 < n_chunks)
            def _():
                idx_start(g + 2, 1).start()

            out_copy(g - 1, 0).wait()
            compose(0)
            out_copy(g + 1, 0).start()
            return c

        lax.fori_loop(0, (n_chunks - 2) // 2, pair_body, 0, unroll=False)

        g_last = n_chunks - 1
        idx_start(g_last, 1).wait()
        out_copy(g_last - 2, 1).wait()
        compose(1)
        out_copy(g_last, 1).start()
        out_copy(g_last - 1, 0).wait()
        out_copy(g_last, 1).wait()

    return k


def kernel(batch_rgn_sqn, encoding):
    b, l = batch_rgn_sqn.shape
    table = encoding[:_TABLE_ROWS, :_PS_DIM].reshape(-1)
    idx = batch_rgn_sqn.reshape(-1).astype(jnp.int32)
    out = _gather_call(b * l, 512)(table, idx)
    return out.reshape(b, l, _PS_DIM)
